# baseline (speedup 1.0000x reference)
import functools

import jax
import jax.numpy as jnp
from jax import lax
from jax.experimental import pallas as pl
from jax.experimental.pallas import tpu as pltpu
from jax.experimental.pallas import tpu_sc as plsc


def _shuffle(v, idx):
    dnums = lax.GatherDimensionNumbers(
        offset_dims=(), collapsed_slice_dims=(0,), start_index_map=(0,)
    )
    return lax.gather(
        v,
        idx[:, None],
        dnums,
        slice_sizes=(1,),
        mode=lax.GatherScatterMode.PROMISE_IN_BOUNDS,
    )


def _argmax_onehot_body(x_hbm, out_hbm, xv, ov):
    @pl.when((lax.axis_index("c") == 0) & (lax.axis_index("s") == 0))
    def _():
        pltpu.sync_copy(x_hbm, xv.at[pl.ds(0, 4)])
        iota = lax.iota(jnp.int32, 16)
        v = jnp.where(iota < 4, xv[...], -jnp.inf)
        m = v
        for d in (1, 2):
            m = jnp.maximum(m, _shuffle(m, iota ^ d))
        w = jnp.where(v == m, iota, 16)
        for d in (1, 2):
            w = jnp.minimum(w, _shuffle(w, iota ^ d))
        ov[...] = jnp.where(iota == w, 1.0, 0.0).astype(jnp.float32)
        pltpu.sync_copy(ov.at[pl.ds(0, 4)], out_hbm)


_argmax_onehot = pl.kernel(
    _argmax_onehot_body,
    out_type=jax.ShapeDtypeStruct((4,), jnp.float32),
    mesh=plsc.VectorSubcoreMesh(
        core_axis_name="c", subcore_axis_name="s", num_cores=1, num_subcores=1
    ),
    scratch_types=[
        pltpu.VMEM((16,), jnp.float32),
        pltpu.VMEM((16,), jnp.float32),
    ],
)


@jax.jit
def kernel(x):
    return _argmax_onehot(x)


# speedup vs baseline: 12.0872x; 12.0872x over previous
import jax
import jax.numpy as jnp
from jax import lax
from jax.experimental import pallas as pl


def _argmax_onehot_body(x_ref, o_ref):
    v = x_ref[...]
    iota = lax.broadcasted_iota(jnp.int32, (4,), 0)
    m = jnp.max(v)
    idx = jnp.min(jnp.where(v == m, iota, 4))
    o_ref[...] = jnp.where(iota == idx, 1.0, 0.0).astype(jnp.float32)


@jax.jit
def kernel(x):
    return pl.pallas_call(
        _argmax_onehot_body,
        out_shape=jax.ShapeDtypeStruct((4,), jnp.float32),
    )(x)
